# baseline (speedup 1.0000x reference)
import functools

import jax
import jax.numpy as jnp
from jax import lax
from jax.experimental import pallas as pl
from jax.experimental.pallas import tpu as pltpu
from jax.experimental.pallas import tpu_sc as plsc

B = 1024
L = 200
EMB = 32
TEMB = 8
OUT_W = 6 * EMB + TEMB
LP = 208
RW = 5 * EMB + TEMB
NW = 32
B_PER_W = B // NW
BLK = 8
NBLK = B_PER_W // BLK
H0 = 104
H1 = L - H0

OFFS = (0, EMB, 2 * EMB, 3 * EMB, 4 * EMB, 5 * EMB)
CVT_OFFS = tuple(range(0, L - 16, 16)) + (L - 16,)

_mesh = plsc.VectorSubcoreMesh(core_axis_name="c", subcore_axis_name="s")
_params = pltpu.CompilerParams(use_tc_tiling_on_sc=False)


def _cvt_idx(idx_f32_v, idx_i32_v, nfeat):
    def cvt_row(r, c2):
        for f in range(nfeat):
            for c in CVT_OFFS:
                idx_i32_v[f, r, pl.ds(c, 16)] = (
                    idx_f32_v[f, r, pl.ds(c, 16)].astype(jnp.int32))
        return c2
    lax.fori_loop(0, BLK, cvt_row, 0)


def _small_body(his_cate, his_shop, his_node, his_product, his_brand,
                his_time, tgt_cate, tgt_shop, tgt_node, tgt_product,
                tgt_brand, tgt_time, cate_table, shop_table, node_table,
                product_table, brand_table, time_table, out,
                idx_f32_v, idx_i32_v, tgt_idx_v, tgt_rows_v, tgt_time_v,
                fb, ft, gsem, wsem):
    wid = lax.axis_index("s") * 2 + lax.axis_index("c")
    b0 = wid * B_PER_W

    his = (his_cate, his_shop, his_node, his_product, his_brand, his_time)
    tgt = (tgt_cate, tgt_shop, tgt_node, tgt_product, tgt_brand, tgt_time)
    tables = (cate_table, shop_table, node_table, product_table, brand_table,
              time_table)

    for f in range(6):
        pltpu.sync_copy(tgt[f].at[pl.ds(b0, B_PER_W)], tgt_idx_v.at[f])
    gcp = [pltpu.async_copy(tables[f].at[tgt_idx_v.at[f]], tgt_rows_v.at[f],
                            gsem) for f in range(5)]
    gcp.append(pltpu.async_copy(tables[5].at[tgt_idx_v.at[5]], tgt_time_v,
                                gsem))
    for c in gcp:
        c.wait()
    for f in range(5):
        pltpu.sync_copy(
            tgt_rows_v.at[f],
            out.at[pl.ds(b0, B_PER_W), 0, pl.ds(OFFS[f], EMB)])
    pltpu.sync_copy(
        tgt_time_v, out.at[pl.ds(b0, B_PER_W), 0, pl.ds(OFFS[5], TEMB)])

    def blk_body(blk, carry):
        bb = b0 + blk * BLK
        for f in range(6):
            pltpu.sync_copy(his[f].at[pl.ds(bb, BLK)], idx_f32_v.at[f])
        _cvt_idx(idx_f32_v, idx_i32_v, 6)
        wb = [None, None]
        for j in range(BLK):
            p = j % 2
            if wb[p] is not None:
                for c in wb[p]:
                    c.wait()
            gc = []
            for f in range(6):
                buf = fb.at[p, f] if f < 5 else ft.at[p]
                gc.append(pltpu.async_copy(
                    tables[f].at[idx_i32_v.at[f, j, pl.ds(0, H0)]],
                    buf.at[pl.ds(0, H0)], gsem))
                gc.append(pltpu.async_copy(
                    tables[f].at[idx_i32_v.at[f, j, pl.ds(H0, H1)]],
                    buf.at[pl.ds(H0, H1)], gsem))
            for c in gc:
                c.wait()
            wb[p] = [pltpu.async_copy(
                fb.at[p, f],
                out.at[bb + j, pl.ds(1, L), pl.ds(OFFS[f], EMB)], wsem)
                for f in range(5)]
            wb[p].append(pltpu.async_copy(
                ft.at[p], out.at[bb + j, pl.ds(1, L), pl.ds(OFFS[5], TEMB)],
                wsem))
        for cs in wb:
            if cs is not None:
                for c in cs:
                    c.wait()
        return carry

    lax.fori_loop(0, NBLK, blk_body, 0)


def _item_body(his_item, tgt_item, item_table, out,
               idx_f32_v, idx_i32_v, tgt_idx_v, tgt_rows_v, fb, gsem, wsem):
    wid = lax.axis_index("s") * 2 + lax.axis_index("c")
    b0 = wid * B_PER_W

    pltpu.sync_copy(tgt_item.at[pl.ds(b0, B_PER_W)], tgt_idx_v)
    pltpu.async_copy(item_table.at[tgt_idx_v], tgt_rows_v, gsem).wait()
    pltpu.sync_copy(tgt_rows_v, out.at[pl.ds(b0, B_PER_W), 0])

    def blk_body(blk, carry):
        bb = b0 + blk * BLK
        pltpu.sync_copy(his_item.at[pl.ds(bb, BLK)], idx_f32_v.at[0])
        _cvt_idx(idx_f32_v, idx_i32_v, 1)
        wb = [None, None]
        for j in range(BLK):
            p = j % 2
            if wb[p] is not None:
                wb[p].wait()
            c0 = pltpu.async_copy(
                item_table.at[idx_i32_v.at[0, j, pl.ds(0, H0)]],
                fb.at[p, pl.ds(0, H0)], gsem)
            c1 = pltpu.async_copy(
                item_table.at[idx_i32_v.at[0, j, pl.ds(H0, H1)]],
                fb.at[p, pl.ds(H0, H1)], gsem)
            c0.wait()
            c1.wait()
            wb[p] = pltpu.async_copy(
                fb.at[p], out.at[bb + j, pl.ds(1, L)], wsem)
        for c in wb:
            if c is not None:
                c.wait()
        return carry

    lax.fori_loop(0, NBLK, blk_body, 0)


_gather_small = functools.partial(
    pl.kernel,
    mesh=_mesh,
    compiler_params=_params,
    out_type=jax.ShapeDtypeStruct((B, LP, RW), jnp.float32),
    scratch_types=[
        pltpu.VMEM((6, BLK, L), jnp.float32),
        pltpu.VMEM((6, BLK, L), jnp.int32),
        pltpu.VMEM((6, B_PER_W), jnp.int32),
        pltpu.VMEM((5, B_PER_W, EMB), jnp.float32),
        pltpu.VMEM((B_PER_W, TEMB), jnp.float32),
        pltpu.VMEM((2, 5, L, EMB), jnp.float32),
        pltpu.VMEM((2, L, TEMB), jnp.float32),
        pltpu.SemaphoreType.DMA,
        pltpu.SemaphoreType.DMA,
    ],
)(_small_body)

_gather_item = functools.partial(
    pl.kernel,
    mesh=_mesh,
    compiler_params=_params,
    out_type=jax.ShapeDtypeStruct((B, LP, EMB), jnp.float32),
    scratch_types=[
        pltpu.VMEM((1, BLK, L), jnp.float32),
        pltpu.VMEM((1, BLK, L), jnp.int32),
        pltpu.VMEM((B_PER_W,), jnp.int32),
        pltpu.VMEM((B_PER_W, EMB), jnp.float32),
        pltpu.VMEM((2, L, EMB), jnp.float32),
        pltpu.SemaphoreType.DMA,
        pltpu.SemaphoreType.DMA,
    ],
)(_item_body)


def kernel(item_id_his_batch_ph, time_id_his_batch_ph, cate_his_batch_ph,
           shop_his_batch_ph, node_his_batch_ph, product_his_batch_ph,
           brand_his_batch_ph, item_id_batch_ph, time_id_batch_ph,
           cate_id_batch_ph, shop_id_batch_ph, node_id_batch_ph,
           product_id_batch_ph, brand_id_batch_ph,
           item_table, cate_table, shop_table, node_table,
           product_table, brand_table, time_table):
    f32 = jnp.float32
    plane_item = _gather_item(
        item_id_his_batch_ph.astype(f32), item_id_batch_ph, item_table)
    plane_rest = _gather_small(
        cate_his_batch_ph.astype(f32), shop_his_batch_ph.astype(f32),
        node_his_batch_ph.astype(f32), product_his_batch_ph.astype(f32),
        brand_his_batch_ph.astype(f32), time_id_his_batch_ph.astype(f32),
        cate_id_batch_ph, shop_id_batch_ph, node_id_batch_ph,
        product_id_batch_ph, brand_id_batch_ph, time_id_batch_ph,
        cate_table, shop_table, node_table, product_table, brand_table,
        time_table,
    )
    return jnp.concatenate(
        [plane_item[:, :L + 1, :], plane_rest[:, :L + 1, :]], axis=2)


# speedup vs baseline: 1.1215x; 1.1215x over previous
import functools

import jax
import jax.numpy as jnp
from jax import lax
from jax.experimental import pallas as pl
from jax.experimental.pallas import tpu as pltpu
from jax.experimental.pallas import tpu_sc as plsc

B = 1024
L = 200
EMB = 32
TEMB = 8
OUT_W = 6 * EMB + TEMB
LP = 208
RW = 5 * EMB + TEMB
RWP = 256
IWP = 128
NW = 32
B_PER_W = B // NW
BLK = 8
NBLK = B_PER_W // BLK
H0 = 104
H1 = L - H0

OFFS = (0, EMB, 2 * EMB, 3 * EMB, 4 * EMB, 5 * EMB)
CVT_OFFS = tuple(range(0, L - 16, 16)) + (L - 16,)

_mesh = plsc.VectorSubcoreMesh(core_axis_name="c", subcore_axis_name="s")
_params = pltpu.CompilerParams(use_tc_tiling_on_sc=False)


def _cvt_idx(idx_f32_v, idx_i32_v, nfeat):
    def cvt_row(r, c2):
        for f in range(nfeat):
            for c in CVT_OFFS:
                idx_i32_v[f, r, pl.ds(c, 16)] = (
                    idx_f32_v[f, r, pl.ds(c, 16)].astype(jnp.int32))
        return c2
    lax.fori_loop(0, BLK, cvt_row, 0)


def _small_body(his_cate, his_shop, his_node, his_product, his_brand,
                his_time, tgt_cate, tgt_shop, tgt_node, tgt_product,
                tgt_brand, tgt_time, cate_table, shop_table, node_table,
                product_table, brand_table, time_table, out,
                idx_f32_v, idx_i32_v, tgt_idx_v, tgt_rows_v, tgt_time_v,
                fb, ft, gsem, wsem):
    wid = lax.axis_index("s") * 2 + lax.axis_index("c")
    b0 = wid * B_PER_W

    his = (his_cate, his_shop, his_node, his_product, his_brand, his_time)
    tgt = (tgt_cate, tgt_shop, tgt_node, tgt_product, tgt_brand, tgt_time)
    tables = (cate_table, shop_table, node_table, product_table, brand_table,
              time_table)

    for f in range(6):
        pltpu.sync_copy(tgt[f].at[pl.ds(b0, B_PER_W)], tgt_idx_v.at[f])
    gcp = [pltpu.async_copy(tables[f].at[tgt_idx_v.at[f]], tgt_rows_v.at[f],
                            gsem) for f in range(5)]
    gcp.append(pltpu.async_copy(tables[5].at[tgt_idx_v.at[5]], tgt_time_v,
                                gsem))
    for c in gcp:
        c.wait()
    for f in range(5):
        pltpu.sync_copy(
            tgt_rows_v.at[f],
            out.at[pl.ds(b0, B_PER_W), 0, pl.ds(OFFS[f], EMB)])
    pltpu.sync_copy(
        tgt_time_v, out.at[pl.ds(b0, B_PER_W), 0, pl.ds(OFFS[5], TEMB)])

    def blk_body(blk, carry):
        bb = b0 + blk * BLK
        for f in range(6):
            pltpu.sync_copy(his[f].at[pl.ds(bb, BLK)], idx_f32_v.at[f])
        _cvt_idx(idx_f32_v, idx_i32_v, 6)
        wb = [None, None]
        for j in range(BLK):
            p = j % 2
            if wb[p] is not None:
                for c in wb[p]:
                    c.wait()
            gc = []
            for f in range(6):
                buf = fb.at[p, f] if f < 5 else ft.at[p]
                gc.append(pltpu.async_copy(
                    tables[f].at[idx_i32_v.at[f, j, pl.ds(0, H0)]],
                    buf.at[pl.ds(0, H0)], gsem))
                gc.append(pltpu.async_copy(
                    tables[f].at[idx_i32_v.at[f, j, pl.ds(H0, H1)]],
                    buf.at[pl.ds(H0, H1)], gsem))
            for c in gc:
                c.wait()
            wb[p] = [pltpu.async_copy(
                fb.at[p, f],
                out.at[bb + j, pl.ds(1, L), pl.ds(OFFS[f], EMB)], wsem)
                for f in range(5)]
            wb[p].append(pltpu.async_copy(
                ft.at[p], out.at[bb + j, pl.ds(1, L), pl.ds(OFFS[5], TEMB)],
                wsem))
        for cs in wb:
            if cs is not None:
                for c in cs:
                    c.wait()
        return carry

    lax.fori_loop(0, NBLK, blk_body, 0)


def _item_body(his_item, tgt_item, item_table, out,
               idx_f32_v, idx_i32_v, tgt_idx_v, tgt_rows_v, fb, gsem, wsem):
    wid = lax.axis_index("s") * 2 + lax.axis_index("c")
    b0 = wid * B_PER_W

    pltpu.sync_copy(tgt_item.at[pl.ds(b0, B_PER_W)], tgt_idx_v)
    pltpu.async_copy(item_table.at[tgt_idx_v], tgt_rows_v, gsem).wait()
    pltpu.sync_copy(tgt_rows_v,
                    out.at[pl.ds(b0, B_PER_W), 0, pl.ds(0, EMB)])

    def blk_body(blk, carry):
        bb = b0 + blk * BLK
        pltpu.sync_copy(his_item.at[pl.ds(bb, BLK)], idx_f32_v.at[0])
        _cvt_idx(idx_f32_v, idx_i32_v, 1)
        wb = [None, None]
        for j in range(BLK):
            p = j % 2
            if wb[p] is not None:
                wb[p].wait()
            c0 = pltpu.async_copy(
                item_table.at[idx_i32_v.at[0, j, pl.ds(0, H0)]],
                fb.at[p, pl.ds(0, H0)], gsem)
            c1 = pltpu.async_copy(
                item_table.at[idx_i32_v.at[0, j, pl.ds(H0, H1)]],
                fb.at[p, pl.ds(H0, H1)], gsem)
            c0.wait()
            c1.wait()
            wb[p] = pltpu.async_copy(
                fb.at[p], out.at[bb + j, pl.ds(1, L), pl.ds(0, EMB)], wsem)
        for c in wb:
            if c is not None:
                c.wait()
        return carry

    lax.fori_loop(0, NBLK, blk_body, 0)


_gather_small = functools.partial(
    pl.kernel,
    mesh=_mesh,
    compiler_params=_params,
    out_type=jax.ShapeDtypeStruct((B, LP, RWP), jnp.float32),
    scratch_types=[
        pltpu.VMEM((6, BLK, L), jnp.float32),
        pltpu.VMEM((6, BLK, L), jnp.int32),
        pltpu.VMEM((6, B_PER_W), jnp.int32),
        pltpu.VMEM((5, B_PER_W, EMB), jnp.float32),
        pltpu.VMEM((B_PER_W, TEMB), jnp.float32),
        pltpu.VMEM((2, 5, L, EMB), jnp.float32),
        pltpu.VMEM((2, L, TEMB), jnp.float32),
        pltpu.SemaphoreType.DMA,
        pltpu.SemaphoreType.DMA,
    ],
)(_small_body)

_gather_item = functools.partial(
    pl.kernel,
    mesh=_mesh,
    compiler_params=_params,
    out_type=jax.ShapeDtypeStruct((B, LP, IWP), jnp.float32),
    scratch_types=[
        pltpu.VMEM((1, BLK, L), jnp.float32),
        pltpu.VMEM((1, BLK, L), jnp.int32),
        pltpu.VMEM((B_PER_W,), jnp.int32),
        pltpu.VMEM((B_PER_W, EMB), jnp.float32),
        pltpu.VMEM((2, L, EMB), jnp.float32),
        pltpu.SemaphoreType.DMA,
        pltpu.SemaphoreType.DMA,
    ],
)(_item_body)


def kernel(item_id_his_batch_ph, time_id_his_batch_ph, cate_his_batch_ph,
           shop_his_batch_ph, node_his_batch_ph, product_his_batch_ph,
           brand_his_batch_ph, item_id_batch_ph, time_id_batch_ph,
           cate_id_batch_ph, shop_id_batch_ph, node_id_batch_ph,
           product_id_batch_ph, brand_id_batch_ph,
           item_table, cate_table, shop_table, node_table,
           product_table, brand_table, time_table):
    f32 = jnp.float32
    plane_item = _gather_item(
        item_id_his_batch_ph.astype(f32), item_id_batch_ph, item_table)
    plane_rest = _gather_small(
        cate_his_batch_ph.astype(f32), shop_his_batch_ph.astype(f32),
        node_his_batch_ph.astype(f32), product_his_batch_ph.astype(f32),
        brand_his_batch_ph.astype(f32), time_id_his_batch_ph.astype(f32),
        cate_id_batch_ph, shop_id_batch_ph, node_id_batch_ph,
        product_id_batch_ph, brand_id_batch_ph, time_id_batch_ph,
        cate_table, shop_table, node_table, product_table, brand_table,
        time_table,
    )
    return jnp.concatenate(
        [plane_item[:, :L + 1, :EMB], plane_rest[:, :L + 1, :RW]], axis=2)


# speedup vs baseline: 1.3328x; 1.1884x over previous
import functools

import jax
import jax.numpy as jnp
from jax import lax
from jax.experimental import pallas as pl
from jax.experimental.pallas import tpu as pltpu
from jax.experimental.pallas import tpu_sc as plsc

B = 1024
L = 200
EMB = 32
TEMB = 8
OUT_W = 6 * EMB + TEMB
LP = 208
IWP = 128
NW = 32
B_PER_W = B // NW
BLK = 8
NBLK = B_PER_W // BLK
H0 = 104
H1 = L - H0

OFFS = (0, EMB, 2 * EMB, 3 * EMB, 0, EMB)
SPLANE = (0, 0, 0, 0, 1, 1)
CVT_OFFS = tuple(range(0, L - 16, 16)) + (L - 16,)

_mesh = plsc.VectorSubcoreMesh(core_axis_name="c", subcore_axis_name="s")
_params = pltpu.CompilerParams(use_tc_tiling_on_sc=False)


def _cvt_idx(idx_f32_v, idx_i32_v, nfeat):
    def cvt_row(r, c2):
        for f in range(nfeat):
            for c in CVT_OFFS:
                idx_i32_v[f, r, pl.ds(c, 16)] = (
                    idx_f32_v[f, r, pl.ds(c, 16)].astype(jnp.int32))
        return c2
    lax.fori_loop(0, BLK, cvt_row, 0)


def _small_body(his_cate, his_shop, his_node, his_product, his_brand,
                his_time, tgt_cate, tgt_shop, tgt_node, tgt_product,
                tgt_brand, tgt_time, cate_table, shop_table, node_table,
                product_table, brand_table, time_table, out1, out2,
                idx_f32_v, idx_i32_v, tgt_idx_v, tgt_rows_v, tgt_time_v,
                fb, ft, gsem, wsem):
    wid = lax.axis_index("s") * 2 + lax.axis_index("c")
    b0 = wid * B_PER_W

    his = (his_cate, his_shop, his_node, his_product, his_brand, his_time)
    tgt = (tgt_cate, tgt_shop, tgt_node, tgt_product, tgt_brand, tgt_time)
    tables = (cate_table, shop_table, node_table, product_table, brand_table,
              time_table)
    outs = (out1, out1, out1, out1, out2, out2)

    for f in range(6):
        pltpu.sync_copy(tgt[f].at[pl.ds(b0, B_PER_W)], tgt_idx_v.at[f])
    gcp = [pltpu.async_copy(tables[f].at[tgt_idx_v.at[f]], tgt_rows_v.at[f],
                            gsem) for f in range(5)]
    gcp.append(pltpu.async_copy(tables[5].at[tgt_idx_v.at[5]], tgt_time_v,
                                gsem))
    for c in gcp:
        c.wait()
    for f in range(5):
        pltpu.sync_copy(
            tgt_rows_v.at[f],
            outs[f].at[pl.ds(b0, B_PER_W), 0, pl.ds(OFFS[f], EMB)])
    pltpu.sync_copy(
        tgt_time_v, out2.at[pl.ds(b0, B_PER_W), 0, pl.ds(OFFS[5], TEMB)])

    def blk_body(blk, carry):
        bb = b0 + blk * BLK
        for f in range(6):
            pltpu.sync_copy(his[f].at[pl.ds(bb, BLK)], idx_f32_v.at[f])
        _cvt_idx(idx_f32_v, idx_i32_v, 6)
        wb = [None, None]
        for j in range(BLK):
            p = j % 2
            if wb[p] is not None:
                for c in wb[p]:
                    c.wait()
            gc = []
            for f in range(6):
                buf = fb.at[p, f] if f < 5 else ft.at[p]
                gc.append(pltpu.async_copy(
                    tables[f].at[idx_i32_v.at[f, j, pl.ds(0, H0)]],
                    buf.at[pl.ds(0, H0)], gsem))
                gc.append(pltpu.async_copy(
                    tables[f].at[idx_i32_v.at[f, j, pl.ds(H0, H1)]],
                    buf.at[pl.ds(H0, H1)], gsem))
            for c in gc:
                c.wait()
            wb[p] = [pltpu.async_copy(
                fb.at[p, f],
                outs[f].at[bb + j, pl.ds(1, L), pl.ds(OFFS[f], EMB)], wsem)
                for f in range(5)]
            wb[p].append(pltpu.async_copy(
                ft.at[p], out2.at[bb + j, pl.ds(1, L), pl.ds(OFFS[5], TEMB)],
                wsem))
        for cs in wb:
            if cs is not None:
                for c in cs:
                    c.wait()
        return carry

    lax.fori_loop(0, NBLK, blk_body, 0)


def _item_body(his_item, tgt_item, item_table, out,
               idx_f32_v, idx_i32_v, tgt_idx_v, tgt_rows_v, fb, gsem, wsem):
    wid = lax.axis_index("s") * 2 + lax.axis_index("c")
    b0 = wid * B_PER_W

    pltpu.sync_copy(tgt_item.at[pl.ds(b0, B_PER_W)], tgt_idx_v)
    pltpu.async_copy(item_table.at[tgt_idx_v], tgt_rows_v, gsem).wait()
    pltpu.sync_copy(tgt_rows_v,
                    out.at[pl.ds(b0, B_PER_W), 0, pl.ds(0, EMB)])

    def blk_body(blk, carry):
        bb = b0 + blk * BLK
        pltpu.sync_copy(his_item.at[pl.ds(bb, BLK)], idx_f32_v.at[0])
        _cvt_idx(idx_f32_v, idx_i32_v, 1)
        wb = [None, None]
        for j in range(BLK):
            p = j % 2
            if wb[p] is not None:
                wb[p].wait()
            c0 = pltpu.async_copy(
                item_table.at[idx_i32_v.at[0, j, pl.ds(0, H0)]],
                fb.at[p, pl.ds(0, H0)], gsem)
            c1 = pltpu.async_copy(
                item_table.at[idx_i32_v.at[0, j, pl.ds(H0, H1)]],
                fb.at[p, pl.ds(H0, H1)], gsem)
            c0.wait()
            c1.wait()
            wb[p] = pltpu.async_copy(
                fb.at[p], out.at[bb + j, pl.ds(1, L), pl.ds(0, EMB)], wsem)
        for c in wb:
            if c is not None:
                c.wait()
        return carry

    lax.fori_loop(0, NBLK, blk_body, 0)


_gather_small = functools.partial(
    pl.kernel,
    mesh=_mesh,
    compiler_params=_params,
    out_type=(jax.ShapeDtypeStruct((B, LP, IWP), jnp.float32),
              jax.ShapeDtypeStruct((B, LP, IWP), jnp.float32)),
    scratch_types=[
        pltpu.VMEM((6, BLK, L), jnp.float32),
        pltpu.VMEM((6, BLK, L), jnp.int32),
        pltpu.VMEM((6, B_PER_W), jnp.int32),
        pltpu.VMEM((5, B_PER_W, EMB), jnp.float32),
        pltpu.VMEM((B_PER_W, TEMB), jnp.float32),
        pltpu.VMEM((2, 5, L, EMB), jnp.float32),
        pltpu.VMEM((2, L, TEMB), jnp.float32),
        pltpu.SemaphoreType.DMA,
        pltpu.SemaphoreType.DMA,
    ],
)(_small_body)

_gather_item = functools.partial(
    pl.kernel,
    mesh=_mesh,
    compiler_params=_params,
    out_type=jax.ShapeDtypeStruct((B, LP, IWP), jnp.float32),
    scratch_types=[
        pltpu.VMEM((1, BLK, L), jnp.float32),
        pltpu.VMEM((1, BLK, L), jnp.int32),
        pltpu.VMEM((B_PER_W,), jnp.int32),
        pltpu.VMEM((B_PER_W, EMB), jnp.float32),
        pltpu.VMEM((2, L, EMB), jnp.float32),
        pltpu.SemaphoreType.DMA,
        pltpu.SemaphoreType.DMA,
    ],
)(_item_body)


def kernel(item_id_his_batch_ph, time_id_his_batch_ph, cate_his_batch_ph,
           shop_his_batch_ph, node_his_batch_ph, product_his_batch_ph,
           brand_his_batch_ph, item_id_batch_ph, time_id_batch_ph,
           cate_id_batch_ph, shop_id_batch_ph, node_id_batch_ph,
           product_id_batch_ph, brand_id_batch_ph,
           item_table, cate_table, shop_table, node_table,
           product_table, brand_table, time_table):
    f32 = jnp.float32
    plane_item = _gather_item(
        item_id_his_batch_ph.astype(f32), item_id_batch_ph, item_table)
    plane_s1, plane_s2 = _gather_small(
        cate_his_batch_ph.astype(f32), shop_his_batch_ph.astype(f32),
        node_his_batch_ph.astype(f32), product_his_batch_ph.astype(f32),
        brand_his_batch_ph.astype(f32), time_id_his_batch_ph.astype(f32),
        cate_id_batch_ph, shop_id_batch_ph, node_id_batch_ph,
        product_id_batch_ph, brand_id_batch_ph, time_id_batch_ph,
        cate_table, shop_table, node_table, product_table, brand_table,
        time_table,
    )
    return jnp.concatenate(
        [plane_item[:, :L + 1, :EMB], plane_s1[:, :L + 1, :],
         plane_s2[:, :L + 1, :EMB + TEMB]], axis=2)
